# baseline (speedup 1.0000x reference)
import functools

import jax
import jax.numpy as jnp
from jax import lax
from jax.experimental import pallas as pl


def _pass1(x_ref, cf_ref, watt_ref, wa_ref, wb_ref, wc_ref, wn_ref, wl_ref,
           wd1_ref, wd2_ref, d_ref, s_ref, q_ref, *, CN, CL):
    f32 = jnp.float32
    xb = x_ref[0]
    cf = cf_ref[0]
    J = CN * CL

    S = (lax.broadcasted_iota(jnp.int32, (J, CN), 0) // CL
         == lax.broadcasted_iota(jnp.int32, (J, CN), 1)).astype(f32)
    S2 = (lax.broadcasted_iota(jnp.int32, (CN, J), 1) // CL
          == lax.broadcasted_iota(jnp.int32, (CN, J), 0)).astype(f32)
    T = (lax.broadcasted_iota(jnp.int32, (J, CL), 0) % CL
         == lax.broadcasted_iota(jnp.int32, (J, CL), 1)).astype(f32)
    T2 = (lax.broadcasted_iota(jnp.int32, (CL, J), 1) % CL
          == lax.broadcasted_iota(jnp.int32, (CL, J), 0)).astype(f32)

    att = jnp.dot(watt_ref[...], cf, preferred_element_type=f32, precision=lax.Precision.HIGHEST)
    e = jnp.exp(att - jnp.max(att))
    den_k = jnp.dot(e, S, preferred_element_type=f32, precision=lax.Precision.HIGHEST)
    den_l = jnp.dot(e, T, preferred_element_type=f32, precision=lax.Precision.HIGHEST)
    soft_last = e / jnp.dot(den_k, S2, preferred_element_type=f32, precision=lax.Precision.HIGHEST)
    soft_pen = e / jnp.dot(den_l, T2, preferred_element_type=f32, precision=lax.Precision.HIGHEST)

    curver_inter = jnp.dot(cf * soft_last, S, preferred_element_type=f32, precision=lax.Precision.HIGHEST)
    curves_intra = jnp.dot(cf * soft_pen, T, preferred_element_type=f32, precision=lax.Precision.HIGHEST)

    CI = jnp.dot(wa_ref[...], curver_inter, preferred_element_type=f32, precision=lax.Precision.HIGHEST)
    CLm = jnp.dot(wb_ref[...], curves_intra, preferred_element_type=f32, precision=lax.Precision.HIGHEST)
    WnCI = jnp.dot(wn_ref[...], CI, preferred_element_type=f32, precision=lax.Precision.HIGHEST)
    WlCL = jnp.dot(wl_ref[...], CLm, preferred_element_type=f32, precision=lax.Precision.HIGHEST)

    A = jnp.dot(wc_ref[...], xb, preferred_element_type=f32, precision=lax.Precision.HIGHEST)

    Li = lax.dot_general(CI, A, (((0,), (0,)), ((), ())),
                         preferred_element_type=f32, precision=lax.Precision.HIGHEST)
    Ei = jnp.exp(Li - jnp.max(Li, axis=0, keepdims=True))
    Pi = Ei / jnp.sum(Ei, axis=0, keepdims=True)
    Xi = jnp.dot(WnCI, Pi, preferred_element_type=f32, precision=lax.Precision.HIGHEST)

    Ll = lax.dot_general(CLm, A, (((0,), (0,)), ((), ())),
                         preferred_element_type=f32, precision=lax.Precision.HIGHEST)
    El = jnp.exp(Ll - jnp.max(Ll, axis=0, keepdims=True))
    Pl = El / jnp.sum(El, axis=0, keepdims=True)
    Xl = jnp.dot(WlCL, Pl, preferred_element_type=f32, precision=lax.Precision.HIGHEST)

    db = (jnp.dot(wd1_ref[...], Xi, preferred_element_type=f32, precision=lax.Precision.HIGHEST)
          + jnp.dot(wd2_ref[...], Xl, preferred_element_type=f32, precision=lax.Precision.HIGHEST))
    d_ref[0] = db
    s_ref[0] = jnp.sum(db, axis=1, keepdims=True)
    q_ref[0] = jnp.sum(db * db, axis=1, keepdims=True)


def _pass2(d_ref, x_ref, s_ref, q_ref, g_ref, b_ref, out_ref, *, count):
    mean = jnp.sum(s_ref[...], axis=1, keepdims=True) / count
    var = jnp.sum(q_ref[...], axis=1, keepdims=True) / count - mean * mean
    scale = g_ref[...] * lax.rsqrt(var + 1e-5)
    shift = b_ref[...] - mean * scale
    y = x_ref[0] + d_ref[0] * scale + shift
    out_ref[0] = jnp.where(y >= 0, y, 0.2 * y)


@jax.jit
def kernel(x, curves, w_att, Wa, Wb, Wc, Wn, Wl, Wd, gamma, beta):
    B, C, N = x.shape
    CN, CL = curves.shape[2], curves.shape[3]
    MID = Wa.shape[0]
    J = CN * CL
    f32 = jnp.float32

    curves_flat = curves.reshape(B, C, J)
    watt2 = w_att.reshape(1, C)
    Wd1 = Wd[:, :MID]
    Wd2 = Wd[:, MID:]

    d, s, q = pl.pallas_call(
        functools.partial(_pass1, CN=CN, CL=CL),
        grid=(B,),
        in_specs=[
            pl.BlockSpec((1, C, N), lambda b: (b, 0, 0)),
            pl.BlockSpec((1, C, J), lambda b: (b, 0, 0)),
            pl.BlockSpec((1, C), lambda b: (0, 0)),
            pl.BlockSpec((MID, C), lambda b: (0, 0)),
            pl.BlockSpec((MID, C), lambda b: (0, 0)),
            pl.BlockSpec((MID, C), lambda b: (0, 0)),
            pl.BlockSpec((MID, MID), lambda b: (0, 0)),
            pl.BlockSpec((MID, MID), lambda b: (0, 0)),
            pl.BlockSpec((C, MID), lambda b: (0, 0)),
            pl.BlockSpec((C, MID), lambda b: (0, 0)),
        ],
        out_specs=[
            pl.BlockSpec((1, C, N), lambda b: (b, 0, 0)),
            pl.BlockSpec((1, C, 1), lambda b: (b, 0, 0)),
            pl.BlockSpec((1, C, 1), lambda b: (b, 0, 0)),
        ],
        out_shape=[
            jax.ShapeDtypeStruct((B, C, N), f32),
            jax.ShapeDtypeStruct((B, C, 1), f32),
            jax.ShapeDtypeStruct((B, C, 1), f32),
        ],
    )(x, curves_flat, watt2, Wa, Wb, Wc, Wn, Wl, Wd1, Wd2)

    s_cb = s[:, :, 0].T
    q_cb = q[:, :, 0].T

    out = pl.pallas_call(
        functools.partial(_pass2, count=float(B * N)),
        grid=(B,),
        in_specs=[
            pl.BlockSpec((1, C, N), lambda b: (b, 0, 0)),
            pl.BlockSpec((1, C, N), lambda b: (b, 0, 0)),
            pl.BlockSpec((C, B), lambda b: (0, 0)),
            pl.BlockSpec((C, B), lambda b: (0, 0)),
            pl.BlockSpec((C, 1), lambda b: (0, 0)),
            pl.BlockSpec((C, 1), lambda b: (0, 0)),
        ],
        out_specs=pl.BlockSpec((1, C, N), lambda b: (b, 0, 0)),
        out_shape=jax.ShapeDtypeStruct((B, C, N), f32),
    )(d, x, s_cb, q_cb, gamma.reshape(C, 1), beta.reshape(C, 1))

    return out


# speedup vs baseline: 1.1409x; 1.1409x over previous
import functools

import jax
import jax.numpy as jnp
from jax import lax
from jax.experimental import pallas as pl
from jax.experimental.pallas import tpu as pltpu


def _pass1(x_ref, cf_ref, watt_ref, wa_ref, wb_ref, wc_ref, wn_ref, wl_ref,
           wd1_ref, wd2_ref, d_ref, s_ref, q_ref, *, CN, CL):
    f32 = jnp.float32
    xb = x_ref[0]
    cf = cf_ref[0]
    J = CN * CL

    S = (lax.broadcasted_iota(jnp.int32, (J, CN), 0) // CL
         == lax.broadcasted_iota(jnp.int32, (J, CN), 1)).astype(f32)
    S2 = (lax.broadcasted_iota(jnp.int32, (CN, J), 1) // CL
          == lax.broadcasted_iota(jnp.int32, (CN, J), 0)).astype(f32)
    T = (lax.broadcasted_iota(jnp.int32, (J, CL), 0) % CL
         == lax.broadcasted_iota(jnp.int32, (J, CL), 1)).astype(f32)
    T2 = (lax.broadcasted_iota(jnp.int32, (CL, J), 1) % CL
          == lax.broadcasted_iota(jnp.int32, (CL, J), 0)).astype(f32)

    att = jnp.dot(watt_ref[...], cf, preferred_element_type=f32, precision=lax.Precision.HIGHEST)
    e = jnp.exp(att - jnp.max(att))
    den_k = jnp.dot(e, S, preferred_element_type=f32, precision=lax.Precision.HIGHEST)
    den_l = jnp.dot(e, T, preferred_element_type=f32, precision=lax.Precision.HIGHEST)
    soft_last = e / jnp.dot(den_k, S2, preferred_element_type=f32, precision=lax.Precision.HIGHEST)
    soft_pen = e / jnp.dot(den_l, T2, preferred_element_type=f32, precision=lax.Precision.HIGHEST)

    curver_inter = jnp.dot(cf * soft_last, S, preferred_element_type=f32, precision=lax.Precision.HIGHEST)
    curves_intra = jnp.dot(cf * soft_pen, T, preferred_element_type=f32, precision=lax.Precision.HIGHEST)

    CI = jnp.dot(wa_ref[...], curver_inter, preferred_element_type=f32, precision=lax.Precision.HIGHEST)
    CLm = jnp.dot(wb_ref[...], curves_intra, preferred_element_type=f32, precision=lax.Precision.HIGHEST)
    WnCI = jnp.dot(wn_ref[...], CI, preferred_element_type=f32, precision=lax.Precision.HIGHEST)
    WlCL = jnp.dot(wl_ref[...], CLm, preferred_element_type=f32, precision=lax.Precision.HIGHEST)

    A = jnp.dot(wc_ref[...], xb, preferred_element_type=f32, precision=lax.Precision.HIGHEST)

    M1 = jnp.dot(wd1_ref[...], WnCI, preferred_element_type=f32, precision=lax.Precision.HIGHEST)
    M2 = jnp.dot(wd2_ref[...], WlCL, preferred_element_type=f32, precision=lax.Precision.HIGHEST)

    Li = lax.dot_general(CI, A, (((0,), (0,)), ((), ())),
                         preferred_element_type=f32, precision=lax.Precision.HIGHEST)
    Ei = jnp.exp(Li - jnp.max(Li, axis=0, keepdims=True))
    Pi = Ei / jnp.sum(Ei, axis=0, keepdims=True)

    Ll = lax.dot_general(CLm, A, (((0,), (0,)), ((), ())),
                         preferred_element_type=f32, precision=lax.Precision.HIGHEST)
    El = jnp.exp(Ll - jnp.max(Ll, axis=0, keepdims=True))
    Pl = El / jnp.sum(El, axis=0, keepdims=True)

    db = (jnp.dot(M1, Pi, preferred_element_type=f32, precision=lax.Precision.HIGHEST)
          + jnp.dot(M2, Pl, preferred_element_type=f32, precision=lax.Precision.HIGHEST))
    d_ref[0] = db.astype(jnp.bfloat16)
    s_ref[0] = jnp.sum(db, axis=1, keepdims=True)
    q_ref[0] = jnp.sum(db * db, axis=1, keepdims=True)


def _pass2(d_ref, x_ref, s_ref, q_ref, g_ref, b_ref, out_ref, *, count):
    mean = jnp.sum(s_ref[...], axis=1, keepdims=True) / count
    var = jnp.sum(q_ref[...], axis=1, keepdims=True) / count - mean * mean
    scale = g_ref[...] * lax.rsqrt(var + 1e-5)
    shift = b_ref[...] - mean * scale
    y = x_ref[0] + d_ref[0].astype(jnp.float32) * scale + shift
    out_ref[0] = jnp.where(y >= 0, y, 0.2 * y)


@jax.jit
def kernel(x, curves, w_att, Wa, Wb, Wc, Wn, Wl, Wd, gamma, beta):
    B, C, N = x.shape
    CN, CL = curves.shape[2], curves.shape[3]
    MID = Wa.shape[0]
    J = CN * CL
    f32 = jnp.float32

    curves_flat = curves.reshape(B, C, J)
    watt2 = w_att.reshape(1, C)
    Wd1 = Wd[:, :MID]
    Wd2 = Wd[:, MID:]

    d, s, q = pl.pallas_call(
        functools.partial(_pass1, CN=CN, CL=CL),
        grid=(B,),
        in_specs=[
            pl.BlockSpec((1, C, N), lambda b: (b, 0, 0)),
            pl.BlockSpec((1, C, J), lambda b: (b, 0, 0)),
            pl.BlockSpec((1, C), lambda b: (0, 0)),
            pl.BlockSpec((MID, C), lambda b: (0, 0)),
            pl.BlockSpec((MID, C), lambda b: (0, 0)),
            pl.BlockSpec((MID, C), lambda b: (0, 0)),
            pl.BlockSpec((MID, MID), lambda b: (0, 0)),
            pl.BlockSpec((MID, MID), lambda b: (0, 0)),
            pl.BlockSpec((C, MID), lambda b: (0, 0)),
            pl.BlockSpec((C, MID), lambda b: (0, 0)),
        ],
        out_specs=[
            pl.BlockSpec((1, C, N), lambda b: (b, 0, 0)),
            pl.BlockSpec((1, C, 1), lambda b: (b, 0, 0)),
            pl.BlockSpec((1, C, 1), lambda b: (b, 0, 0)),
        ],
        out_shape=[
            jax.ShapeDtypeStruct((B, C, N), jnp.bfloat16),
            jax.ShapeDtypeStruct((B, C, 1), f32),
            jax.ShapeDtypeStruct((B, C, 1), f32),
        ],
        compiler_params=pltpu.CompilerParams(
            dimension_semantics=("parallel",)),
    )(x, curves_flat, watt2, Wa, Wb, Wc, Wn, Wl, Wd1, Wd2)

    s_cb = s[:, :, 0].T
    q_cb = q[:, :, 0].T

    out = pl.pallas_call(
        functools.partial(_pass2, count=float(B * N)),
        grid=(B,),
        in_specs=[
            pl.BlockSpec((1, C, N), lambda b: (b, 0, 0)),
            pl.BlockSpec((1, C, N), lambda b: (b, 0, 0)),
            pl.BlockSpec((C, B), lambda b: (0, 0)),
            pl.BlockSpec((C, B), lambda b: (0, 0)),
            pl.BlockSpec((C, 1), lambda b: (0, 0)),
            pl.BlockSpec((C, 1), lambda b: (0, 0)),
        ],
        out_specs=pl.BlockSpec((1, C, N), lambda b: (b, 0, 0)),
        out_shape=jax.ShapeDtypeStruct((B, C, N), f32),
        compiler_params=pltpu.CompilerParams(
            dimension_semantics=("parallel",)),
    )(d, x, s_cb, q_cb, gamma.reshape(C, 1), beta.reshape(C, 1))

    return out
